# baseline (speedup 1.0000x reference)
import jax
import jax.numpy as jnp
from jax import lax
from jax.experimental import pallas as pl
from jax.experimental.pallas import tpu as pltpu
from jax.experimental.pallas import tpu_sc as plsc

_NC = 2
_NS = 16
_L = 16


def _sc_gather_body(emb_hbm, idx_hbm, row_hbm, idx_v):
    wid = lax.axis_index("s") * _NC + lax.axis_index("c")

    @pl.when(wid == 0)
    def _():
        pltpu.sync_copy(idx_hbm, idx_v)
        i = idx_v[...][0]
        pltpu.sync_copy(emb_hbm.at[pl.ds(i, 1)], row_hbm)


def _tc_add_body(x_ref, row_ref, o_ref):
    o_ref[...] = x_ref[...] + row_ref[...]


def kernel(x, type_emb, index):
    B, S, D = x.shape
    N = B * S
    xf = x.reshape(N, D)
    idx = jnp.broadcast_to(jnp.asarray(index, jnp.int32), (_L,))

    mesh = plsc.VectorSubcoreMesh(core_axis_name="c", subcore_axis_name="s")
    row = pl.kernel(
        _sc_gather_body,
        out_type=jax.ShapeDtypeStruct((1, D), jnp.float32),
        mesh=mesh,
        scratch_types=[pltpu.VMEM((_L,), jnp.int32)],
    )(type_emb, idx)

    BM = 2048
    out = pl.pallas_call(
        _tc_add_body,
        grid=(N // BM,),
        in_specs=[
            pl.BlockSpec((BM, D), lambda i: (i, 0)),
            pl.BlockSpec((1, D), lambda i: (0, 0)),
        ],
        out_specs=pl.BlockSpec((BM, D), lambda i: (i, 0)),
        out_shape=jax.ShapeDtypeStruct((N, D), x.dtype),
    )(xf, row)
    return out.reshape(B, S, D)
